# baseline (device time: 219578 ns/iter reference)
import jax
import jax.numpy as jnp
from jax import lax
from jax.experimental import pallas as pl
from jax.experimental.pallas import tpu as pltpu

N_LOCAL_EXPERTS = 4
BLK = 512
OH_LANES = 128


def _fused_moe_a2a(x16, oh_keep, oh_send, W1_16, W2_16):
    tokens, d = x16.shape
    n_e, _, f = W1_16.shape
    nblk = tokens // BLK

    def body(x_ref, ohk_ref, ohs_ref, w1_ref, w2_ref, out_ref,
             xp_ref, ohr_ref, ppart_ref, rpart_ref,
             in_sems, blk_send_sems, blk_recv_sems):
        mx = lax.axis_index("x")
        my = lax.axis_index("y")
        mz = lax.axis_index("z")
        peer = (1 - mx, my, mz)

        barrier = pltpu.get_barrier_semaphore()
        pl.semaphore_signal(
            barrier, inc=1, device_id=peer,
            device_id_type=pl.DeviceIdType.MESH,
        )
        pl.semaphore_wait(barrier, 1)

        rdma_x = pltpu.make_async_remote_copy(
            src_ref=x_ref, dst_ref=xp_ref,
            send_sem=in_sems.at[0], recv_sem=in_sems.at[1],
            device_id=peer, device_id_type=pl.DeviceIdType.MESH,
        )
        rdma_oh = pltpu.make_async_remote_copy(
            src_ref=ohs_ref, dst_ref=ohr_ref,
            send_sem=in_sems.at[2], recv_sem=in_sems.at[3],
            device_id=peer, device_id_type=pl.DeviceIdType.MESH,
        )
        rdma_x.start()
        rdma_oh.start()

        def ffn_block(xb, oh_ref_, rows):
            acc = jnp.zeros((BLK, d), jnp.float32)
            for e in range(n_e):
                h = jnp.maximum(
                    jnp.dot(xb, w1_ref[e], preferred_element_type=jnp.float32),
                    0.0,
                ).astype(jnp.bfloat16)
                y = jnp.dot(h, w2_ref[e], preferred_element_type=jnp.float32)
                acc = acc + y * oh_ref_[rows, e:e + 1].astype(jnp.float32)
            return acc

        def my_blk(i, _):
            rows = pl.ds(i * BLK, BLK)
            out_ref[rows, :] = ffn_block(x_ref[rows, :], ohk_ref, rows)
            return 0

        lax.fori_loop(0, nblk, my_blk, 0)

        rdma_x.wait()
        rdma_oh.wait()

        def peer_blk(i, _):
            rows = pl.ds(i * BLK, BLK)
            ppart_ref[rows, :] = ffn_block(
                xp_ref[rows, :], ohr_ref, rows
            ).astype(jnp.bfloat16)
            send = pltpu.make_async_remote_copy(
                src_ref=ppart_ref.at[rows, :],
                dst_ref=rpart_ref.at[rows, :],
                send_sem=blk_send_sems.at[i],
                recv_sem=blk_recv_sems.at[i],
                device_id=peer, device_id_type=pl.DeviceIdType.MESH,
            )
            send.start()
            return 0

        lax.fori_loop(0, nblk, peer_blk, 0)

        def add_blk(i, _):
            rows = pl.ds(i * BLK, BLK)
            done = pltpu.make_async_remote_copy(
                src_ref=ppart_ref.at[rows, :],
                dst_ref=rpart_ref.at[rows, :],
                send_sem=blk_send_sems.at[i],
                recv_sem=blk_recv_sems.at[i],
                device_id=peer, device_id_type=pl.DeviceIdType.MESH,
            )
            done.wait()
            out_ref[rows, :] += rpart_ref[rows, :].astype(jnp.float32)
            return 0

        lax.fori_loop(0, nblk, add_blk, 0)

    return pl.pallas_call(
        body,
        out_shape=jax.ShapeDtypeStruct((tokens, d), jnp.float32),
        in_specs=[pl.BlockSpec(memory_space=pltpu.VMEM)] * 5,
        out_specs=pl.BlockSpec(memory_space=pltpu.VMEM),
        scratch_shapes=[
            pltpu.VMEM((tokens, d), jnp.bfloat16),
            pltpu.VMEM((tokens, OH_LANES), jnp.bfloat16),
            pltpu.VMEM((tokens, d), jnp.bfloat16),
            pltpu.VMEM((tokens, d), jnp.bfloat16),
            pltpu.SemaphoreType.DMA((4,)),
            pltpu.SemaphoreType.DMA((8,)),
            pltpu.SemaphoreType.DMA((8,)),
        ],
        compiler_params=pltpu.CompilerParams(
            collective_id=0, vmem_limit_bytes=100 * 1024 * 1024
        ),
    )(x16, oh_keep, oh_send, W1_16, W2_16)


def kernel(x, assign, W1, W2):
    mx = lax.axis_index("x")

    e_mine = N_LOCAL_EXPERTS * mx + jnp.arange(OH_LANES, dtype=jnp.int32)
    e_peer = N_LOCAL_EXPERTS * (1 - mx) + jnp.arange(OH_LANES, dtype=jnp.int32)
    oh_keep = (assign[:, None] == e_mine[None, :]).astype(jnp.bfloat16)
    oh_send = (assign[:, None] == e_peer[None, :]).astype(jnp.bfloat16)

    return _fused_moe_a2a(
        x.astype(jnp.bfloat16),
        oh_keep,
        oh_send,
        W1.astype(jnp.bfloat16),
        W2.astype(jnp.bfloat16),
    )


# device time: 216711 ns/iter; 1.0132x vs baseline; 1.0132x over previous
import jax
import jax.numpy as jnp
from jax import lax
from jax.experimental import pallas as pl
from jax.experimental.pallas import tpu as pltpu

N_LOCAL_EXPERTS = 4
BLK = 256
OH_LANES = 128


def _fused_moe_a2a(x16, oh_keep, oh_send, W1_16, W2_16):
    tokens, d = x16.shape
    n_e, _, f = W1_16.shape
    nblk = tokens // BLK

    def body(x_ref, ohk_ref, ohs_ref, w1_ref, w2_ref, out_ref,
             xp_ref, ohr_ref, ppart_ref, rpart_ref,
             in_sems, blk_send_sems, blk_recv_sems):
        mx = lax.axis_index("x")
        my = lax.axis_index("y")
        mz = lax.axis_index("z")
        peer = (1 - mx, my, mz)

        barrier = pltpu.get_barrier_semaphore()
        pl.semaphore_signal(
            barrier, inc=1, device_id=peer,
            device_id_type=pl.DeviceIdType.MESH,
        )
        pl.semaphore_wait(barrier, 1)

        rdma_x = pltpu.make_async_remote_copy(
            src_ref=x_ref, dst_ref=xp_ref,
            send_sem=in_sems.at[0], recv_sem=in_sems.at[1],
            device_id=peer, device_id_type=pl.DeviceIdType.MESH,
        )
        rdma_oh = pltpu.make_async_remote_copy(
            src_ref=ohs_ref, dst_ref=ohr_ref,
            send_sem=in_sems.at[2], recv_sem=in_sems.at[3],
            device_id=peer, device_id_type=pl.DeviceIdType.MESH,
        )
        rdma_x.start()
        rdma_oh.start()

        def ffn_block(xb, oh_ref_, rows):
            acc = jnp.zeros((BLK, d), jnp.float32)
            for e in range(n_e):
                h = jnp.maximum(
                    jnp.dot(xb, w1_ref[e], preferred_element_type=jnp.float32),
                    0.0,
                ).astype(jnp.bfloat16)
                y = jnp.dot(h, w2_ref[e], preferred_element_type=jnp.float32)
                acc = acc + y * oh_ref_[rows, e:e + 1].astype(jnp.float32)
            return acc

        def my_blk(i, _):
            rows = pl.ds(i * BLK, BLK)
            out_ref[rows, :] = ffn_block(x_ref[rows, :], ohk_ref, rows)
            return 0

        lax.fori_loop(0, nblk, my_blk, 0)

        rdma_x.wait()
        rdma_oh.wait()

        def peer_blk(i, _):
            rows = pl.ds(i * BLK, BLK)
            ppart_ref[rows, :] = ffn_block(
                xp_ref[rows, :], ohr_ref, rows
            ).astype(jnp.bfloat16)
            send = pltpu.make_async_remote_copy(
                src_ref=ppart_ref.at[rows, :],
                dst_ref=rpart_ref.at[rows, :],
                send_sem=blk_send_sems.at[i],
                recv_sem=blk_recv_sems.at[i],
                device_id=peer, device_id_type=pl.DeviceIdType.MESH,
            )
            send.start()
            return 0

        lax.fori_loop(0, nblk, peer_blk, 0)

        def add_blk(i, _):
            rows = pl.ds(i * BLK, BLK)
            done = pltpu.make_async_remote_copy(
                src_ref=ppart_ref.at[rows, :],
                dst_ref=rpart_ref.at[rows, :],
                send_sem=blk_send_sems.at[i],
                recv_sem=blk_recv_sems.at[i],
                device_id=peer, device_id_type=pl.DeviceIdType.MESH,
            )
            done.wait()
            out_ref[rows, :] += rpart_ref[rows, :].astype(jnp.float32)
            return 0

        lax.fori_loop(0, nblk, add_blk, 0)

    return pl.pallas_call(
        body,
        out_shape=jax.ShapeDtypeStruct((tokens, d), jnp.float32),
        in_specs=[pl.BlockSpec(memory_space=pltpu.VMEM)] * 5,
        out_specs=pl.BlockSpec(memory_space=pltpu.VMEM),
        scratch_shapes=[
            pltpu.VMEM((tokens, d), jnp.bfloat16),
            pltpu.VMEM((tokens, OH_LANES), jnp.bfloat16),
            pltpu.VMEM((tokens, d), jnp.bfloat16),
            pltpu.VMEM((tokens, d), jnp.bfloat16),
            pltpu.SemaphoreType.DMA((4,)),
            pltpu.SemaphoreType.DMA((8,)),
            pltpu.SemaphoreType.DMA((8,)),
        ],
        compiler_params=pltpu.CompilerParams(
            collective_id=0, vmem_limit_bytes=100 * 1024 * 1024
        ),
    )(x16, oh_keep, oh_send, W1_16, W2_16)


def kernel(x, assign, W1, W2):
    mx = lax.axis_index("x")

    e_mine = N_LOCAL_EXPERTS * mx + jnp.arange(OH_LANES, dtype=jnp.int32)
    e_peer = N_LOCAL_EXPERTS * (1 - mx) + jnp.arange(OH_LANES, dtype=jnp.int32)
    oh_keep = (assign[:, None] == e_mine[None, :]).astype(jnp.bfloat16)
    oh_send = (assign[:, None] == e_peer[None, :]).astype(jnp.bfloat16)

    return _fused_moe_a2a(
        x.astype(jnp.bfloat16),
        oh_keep,
        oh_send,
        W1.astype(jnp.bfloat16),
        W2.astype(jnp.bfloat16),
    )
